# initial kernel scaffold (unmeasured)
import jax
import jax.numpy as jnp
from jax import lax
from jax.experimental import pallas as pl
from jax.experimental.pallas import tpu as pltpu


def kernel(
    x,
):
    def body(*refs):
        pass

    out_shape = jax.ShapeDtypeStruct(..., jnp.float32)
    return pl.pallas_call(body, out_shape=out_shape)(...)


# baseline (device time: 103641 ns/iter reference)
import jax
import jax.numpy as jnp
from jax import lax
from jax.experimental import pallas as pl
from jax.experimental.pallas import tpu as pltpu


def kernel(x):
    m, n = x.shape

    def body(x_ref, out_ref, xbf_ref, xrecv_ref, red_ref,
             sem1s, sem1r, sem2s, sem2r):
        mx = lax.axis_index("x")
        my = lax.axis_index("y")

        barrier = pltpu.get_barrier_semaphore()
        pl.semaphore_signal(barrier, inc=1, device_id=(1 - mx, my),
                            device_id_type=pl.DeviceIdType.MESH)
        pl.semaphore_signal(barrier, inc=1, device_id=(mx, 1 - my),
                            device_id_type=pl.DeviceIdType.MESH)
        pl.semaphore_wait(barrier, 2)

        xbf_ref[...] = x_ref[...].astype(jnp.bfloat16)

        rdma1 = pltpu.make_async_remote_copy(
            src_ref=xbf_ref,
            dst_ref=xrecv_ref,
            send_sem=sem1s,
            recv_sem=sem1r,
            device_id=(1 - mx, my),
            device_id_type=pl.DeviceIdType.MESH,
        )
        rdma1.start()
        rdma1.wait()

        red_ref[...] = xbf_ref[...] + xrecv_ref[...]
        out_ref[:, pl.ds(my * n, n)] = red_ref[...]

        rdma2 = pltpu.make_async_remote_copy(
            src_ref=red_ref,
            dst_ref=out_ref.at[:, pl.ds(my * n, n)],
            send_sem=sem2s,
            recv_sem=sem2r,
            device_id=(mx, 1 - my),
            device_id_type=pl.DeviceIdType.MESH,
        )
        rdma2.start()
        rdma2.wait()

    return pl.pallas_call(
        body,
        out_shape=jax.ShapeDtypeStruct((m, 2 * n), jnp.bfloat16),
        in_specs=[pl.BlockSpec(memory_space=pltpu.VMEM)],
        out_specs=pl.BlockSpec(memory_space=pltpu.VMEM),
        scratch_shapes=[
            pltpu.VMEM((m, n), jnp.bfloat16),
            pltpu.VMEM((m, n), jnp.bfloat16),
            pltpu.VMEM((m, n), jnp.bfloat16),
            pltpu.SemaphoreType.DMA,
            pltpu.SemaphoreType.DMA,
            pltpu.SemaphoreType.DMA,
            pltpu.SemaphoreType.DMA,
        ],
        compiler_params=pltpu.CompilerParams(collective_id=0),
    )(x)


# device time: 63719 ns/iter; 1.6265x vs baseline; 1.6265x over previous
import jax
import jax.numpy as jnp
from jax import lax
from jax.experimental import pallas as pl
from jax.experimental.pallas import tpu as pltpu


C = 8


def kernel(x):
    m, n = x.shape
    mc = m // C

    def body(x_ref, out_ref, xbf_ref, xrecv_ref, red_ref,
             sem1s, sem1r, sem2s, sem2r):
        mx = lax.axis_index("x")
        my = lax.axis_index("y")

        barrier = pltpu.get_barrier_semaphore()
        pl.semaphore_signal(barrier, inc=1, device_id=(1 - mx, my),
                            device_id_type=pl.DeviceIdType.MESH)
        pl.semaphore_signal(barrier, inc=1, device_id=(mx, 1 - my),
                            device_id_type=pl.DeviceIdType.MESH)
        pl.semaphore_wait(barrier, 2)

        rdma1 = []
        for c in range(C):
            rows = pl.ds(c * mc, mc)
            xbf_ref[rows, :] = x_ref[rows, :].astype(jnp.bfloat16)
            r = pltpu.make_async_remote_copy(
                src_ref=xbf_ref.at[rows],
                dst_ref=xrecv_ref.at[rows],
                send_sem=sem1s.at[c],
                recv_sem=sem1r.at[c],
                device_id=(1 - mx, my),
                device_id_type=pl.DeviceIdType.MESH,
            )
            r.start()
            rdma1.append(r)

        rdma2 = []
        for c in range(C):
            rows = pl.ds(c * mc, mc)
            rdma1[c].wait_recv()
            red_ref[rows, :] = xbf_ref[rows, :] + xrecv_ref[rows, :]
            r = pltpu.make_async_remote_copy(
                src_ref=red_ref.at[rows],
                dst_ref=out_ref.at[rows, pl.ds(my * n, n)],
                send_sem=sem2s.at[c],
                recv_sem=sem2r.at[c],
                device_id=(mx, 1 - my),
                device_id_type=pl.DeviceIdType.MESH,
            )
            r.start()
            rdma2.append(r)
            out_ref[rows, pl.ds(my * n, n)] = red_ref[rows, :]

        for c in range(C):
            rdma1[c].wait_send()
            rdma2[c].wait()

    return pl.pallas_call(
        body,
        out_shape=jax.ShapeDtypeStruct((m, 2 * n), jnp.bfloat16),
        in_specs=[pl.BlockSpec(memory_space=pltpu.VMEM)],
        out_specs=pl.BlockSpec(memory_space=pltpu.VMEM),
        scratch_shapes=[
            pltpu.VMEM((m, n), jnp.bfloat16),
            pltpu.VMEM((m, n), jnp.bfloat16),
            pltpu.VMEM((m, n), jnp.bfloat16),
            pltpu.SemaphoreType.DMA((C,)),
            pltpu.SemaphoreType.DMA((C,)),
            pltpu.SemaphoreType.DMA((C,)),
            pltpu.SemaphoreType.DMA((C,)),
        ],
        compiler_params=pltpu.CompilerParams(collective_id=0),
    )(x)


# device time: 61113 ns/iter; 1.6959x vs baseline; 1.0426x over previous
import jax
import jax.numpy as jnp
from jax import lax
from jax.experimental import pallas as pl
from jax.experimental.pallas import tpu as pltpu


C = 16


def kernel(x):
    m, n = x.shape
    mc = m // C

    def body(x_ref, out_ref, xbf_ref, xrecv_ref, red_ref,
             sem1s, sem1r, sem2s, sem2r):
        mx = lax.axis_index("x")
        my = lax.axis_index("y")

        barrier = pltpu.get_barrier_semaphore()
        pl.semaphore_signal(barrier, inc=1, device_id=(1 - mx, my),
                            device_id_type=pl.DeviceIdType.MESH)
        pl.semaphore_signal(barrier, inc=1, device_id=(mx, 1 - my),
                            device_id_type=pl.DeviceIdType.MESH)
        pl.semaphore_wait(barrier, 2)

        rdma1 = []
        for c in range(C):
            rows = pl.ds(c * mc, mc)
            xbf_ref[rows, :] = x_ref[rows, :].astype(jnp.bfloat16)
            r = pltpu.make_async_remote_copy(
                src_ref=xbf_ref.at[rows],
                dst_ref=xrecv_ref.at[rows],
                send_sem=sem1s.at[c],
                recv_sem=sem1r.at[c],
                device_id=(1 - mx, my),
                device_id_type=pl.DeviceIdType.MESH,
            )
            r.start()
            rdma1.append(r)

        rdma2 = []
        for c in range(C):
            rows = pl.ds(c * mc, mc)
            rdma1[c].wait_recv()
            red_ref[rows, :] = xbf_ref[rows, :] + xrecv_ref[rows, :]
            r = pltpu.make_async_remote_copy(
                src_ref=red_ref.at[rows],
                dst_ref=out_ref.at[rows, pl.ds(my * n, n)],
                send_sem=sem2s.at[c],
                recv_sem=sem2r.at[c],
                device_id=(mx, 1 - my),
                device_id_type=pl.DeviceIdType.MESH,
            )
            r.start()
            rdma2.append(r)
            out_ref[rows, pl.ds(my * n, n)] = red_ref[rows, :]

        for c in range(C):
            rdma1[c].wait_send()
            rdma2[c].wait()

    return pl.pallas_call(
        body,
        out_shape=jax.ShapeDtypeStruct((m, 2 * n), jnp.bfloat16),
        in_specs=[pl.BlockSpec(memory_space=pltpu.VMEM)],
        out_specs=pl.BlockSpec(memory_space=pltpu.VMEM),
        scratch_shapes=[
            pltpu.VMEM((m, n), jnp.bfloat16),
            pltpu.VMEM((m, n), jnp.bfloat16),
            pltpu.VMEM((m, n), jnp.bfloat16),
            pltpu.SemaphoreType.DMA((C,)),
            pltpu.SemaphoreType.DMA((C,)),
            pltpu.SemaphoreType.DMA((C,)),
            pltpu.SemaphoreType.DMA((C,)),
        ],
        compiler_params=pltpu.CompilerParams(collective_id=0),
    )(x)
